# baseline (speedup 1.0000x reference)
import functools

import jax
import jax.numpy as jnp
from jax import lax
from jax.experimental import pallas as pl
from jax.experimental.pallas import tpu as pltpu
from jax.experimental.pallas import tpu_sc as plsc

N_NODES = 10000
N_EDGES = 320000
D = 128

NC = 2
NS = 16
NW = NC * NS

CHUNK = 128
NCHUNK = 80
EDGES_PER_W = NCHUNK * CHUNK
NBUF = 2
LAG = 1
PF = 2
NIR = 4
UNROLL = 4

N_PAD = 10240
ROWS_PER_TILE = N_PAD // NS


def _matmul_norm_kernel(h_ref, w_ref, norm_ref, out_ref):
    out_ref[...] = (
        jnp.dot(h_ref[...], w_ref[...], preferred_element_type=jnp.float32)
        * norm_ref[...]
    )


def _matmul_norm(h, W, norm):
    nb = 10
    bs = N_NODES // nb
    return pl.pallas_call(
        _matmul_norm_kernel,
        grid=(nb,),
        in_specs=[
            pl.BlockSpec((bs, D), lambda i: (i, 0)),
            pl.BlockSpec((D, D), lambda i: (0, 0)),
            pl.BlockSpec((bs, 1), lambda i: (i, 0)),
        ],
        out_specs=pl.BlockSpec((bs, D), lambda i: (i, 0)),
        out_shape=jax.ShapeDtypeStruct((N_NODES, D), jnp.float32),
    )(h, W, norm)


def _finish_kernel(p0_ref, p1_ref, b_ref, out_ref):
    out_ref[...] = jnp.maximum(p0_ref[...] + p1_ref[...] + b_ref[...], 0.0)


def _finish(p0, p1, b):
    nb = 10
    bs = N_NODES // nb
    return pl.pallas_call(
        _finish_kernel,
        grid=(nb,),
        in_specs=[
            pl.BlockSpec((bs, D), lambda i: (i, 0)),
            pl.BlockSpec((bs, D), lambda i: (i, 0)),
            pl.BlockSpec((1, D), lambda i: (0, 0)),
        ],
        out_specs=pl.BlockSpec((bs, D), lambda i: (i, 0)),
        out_shape=jax.ShapeDtypeStruct((N_NODES, D), jnp.float32),
    )(p0, p1, b)


def _sc_scatter(normh, idx, zeros):
    mesh = plsc.VectorSubcoreMesh(core_axis_name="c", subcore_axis_name="s")

    @functools.partial(
        pl.kernel,
        mesh=mesh,
        out_type=jax.ShapeDtypeStruct((NC, N_PAD, D), jnp.float32),
        scratch_types=[
            [pltpu.VMEM((2, CHUNK), jnp.int32) for _ in range(NIR)],
            [pltpu.VMEM((CHUNK, D), jnp.float32) for _ in range(NBUF)],
            pltpu.VMEM_SHARED((N_PAD, D), jnp.float32),
            [pltpu.SemaphoreType.DMA for _ in range(NIR)],
            [pltpu.SemaphoreType.DMA for _ in range(NBUF)],
            [pltpu.SemaphoreType.DMA for _ in range(NBUF)],
        ],
    )
    def k(normh_hbm, idx_hbm, zeros_hbm, out_hbm,
          islot, rows, acc, isem, gsem, ssem):
        cid = lax.axis_index("c")
        sid = lax.axis_index("s")
        wid = cid * NS + sid

        pltpu.sync_copy(
            zeros_hbm.at[pl.ds(sid * ROWS_PER_TILE, ROWS_PER_TILE)],
            acc.at[pl.ds(sid * ROWS_PER_TILE, ROWS_PER_TILE)],
        )
        plsc.subcore_barrier()

        def idx_start(j, s):
            pltpu.async_copy(idx_hbm.at[wid, j], islot[s], isem[s])

        def idx_wait(s):
            pltpu.make_async_copy(idx_hbm.at[0, 0], islot[s], isem[s]).wait()

        def gather_start(j_unused, s, b):
            pltpu.async_copy(normh_hbm.at[islot[s].at[0]], rows[b], gsem[b])

        def gather_wait(s, b):
            pltpu.make_async_copy(normh_hbm.at[islot[s].at[0]], rows[b],
                                  gsem[b]).wait()

        def scatter_start(j_unused, s, b):
            pltpu.async_copy(rows[b], acc.at[islot[s].at[1]], ssem[b],
                             add=True)

        def scatter_wait(s, b):
            pltpu.make_async_copy(rows[b], acc.at[islot[s].at[1]],
                                  ssem[b]).wait()

        def step(j, jm, do_swait, do_istart, do_tail):
            b = jm % NBUF
            s = jm % NIR
            if do_swait:
                scatter_wait((jm - NBUF) % NIR, b)
            if do_istart:
                idx_start(j + PF, (jm + PF) % NIR)
            idx_wait(s)
            gather_start(j, s, b)
            if do_tail:
                scatter_start(j - LAG, (jm - LAG) % NIR, (jm - LAG) % NBUF)

        idx_start(0, 0)
        idx_start(1, 1)
        for j in range(UNROLL):
            b = j % NBUF
            s = j % NIR
            if j >= NBUF:
                scatter_wait((j - NBUF) % NIR, b)
            idx_start(j + PF, (j + PF) % NIR)
            idx_wait(s)
            gather_start(j, s, b)
            if j >= LAG:
                gather_wait((j - LAG) % NIR, (j - LAG) % NBUF)
                scatter_start(j - LAG, (j - LAG) % NIR, (j - LAG) % NBUF)

        def steady(i, carry):
            for u in range(UNROLL):
                j = i * UNROLL + u
                b = u % NBUF
                s = u % NIR
                scatter_wait((u - NBUF) % NIR, b)
                idx_start(j + PF, (u + PF) % NIR)
                idx_wait(s)
                gather_start(j, s, b)
                gather_wait((u - LAG) % NIR, (u - LAG) % NBUF)
                scatter_start(j - LAG, (u - LAG) % NIR, (u - LAG) % NBUF)
            return carry

        lax.fori_loop(1, NCHUNK // UNROLL - 1, steady, 0)

        for j in range(NCHUNK - UNROLL, NCHUNK):
            u = j % UNROLL
            b = u % NBUF
            s = u % NIR
            scatter_wait((u - NBUF) % NIR, b)
            if j + PF < NCHUNK:
                idx_start(j + PF, (u + PF) % NIR)
            idx_wait(s)
            gather_start(j, s, b)
            gather_wait((u - LAG) % NIR, (u - LAG) % NBUF)
            scatter_start(j - LAG, (u - LAG) % NIR, (u - LAG) % NBUF)

        for j in range(NCHUNK, NCHUNK + LAG):
            u = j % UNROLL
            gather_wait((u - LAG) % NIR, (u - LAG) % NBUF)
            scatter_start(j - LAG, (u - LAG) % NIR, (u - LAG) % NBUF)
        for j in range(NCHUNK - NBUF, NCHUNK):
            u = j % UNROLL
            scatter_wait(u % NIR, u % NBUF)

        plsc.subcore_barrier()
        pltpu.sync_copy(
            acc.at[pl.ds(sid * ROWS_PER_TILE, ROWS_PER_TILE)],
            out_hbm.at[cid, pl.ds(sid * ROWS_PER_TILE, ROWS_PER_TILE)],
        )

    return k(normh, idx, zeros)


def kernel(h, edge_index, norm, W, b):
    normh = _matmul_norm(h, W, norm)
    ppw = EDGES_PER_W - N_EDGES // NW
    src_w = edge_index[0].reshape(NW, N_EDGES // NW)
    dst_w = edge_index[1].reshape(NW, N_EDGES // NW)
    src_pad = jnp.zeros((NW, ppw), jnp.int32)
    dst_pad = jnp.broadcast_to(
        (N_NODES + jnp.arange(NW, dtype=jnp.int32))[:, None], (NW, ppw))
    src_r = jnp.concatenate([src_w, src_pad], axis=1).reshape(NW, NCHUNK, CHUNK)
    dst_r = jnp.concatenate([dst_w, dst_pad], axis=1).reshape(NW, NCHUNK, CHUNK)
    idx = jnp.stack([src_r, dst_r], axis=2)
    zeros = jnp.zeros((N_PAD, D), jnp.float32)
    partials = _sc_scatter(normh, idx, zeros)
    return _finish(partials[0, :N_NODES], partials[1, :N_NODES], b.reshape(1, D))


# speedup vs baseline: 1.4843x; 1.4843x over previous
import functools

import jax
import jax.numpy as jnp
from jax import lax
from jax.experimental import pallas as pl
from jax.experimental.pallas import tpu as pltpu
from jax.experimental.pallas import tpu_sc as plsc

N_NODES = 10000
N_EDGES = 320000
D = 128

NC = 2
NS = 16
NW = NC * NS

CHUNK = 80
NCHUNK = 126
EDGES_PER_W = NCHUNK * CHUNK
NBUF = 2

N_PAD = 10240
ROWS_PER_TILE = N_PAD // NS


def _matmul_norm_kernel(h_ref, w_ref, norm_ref, out_ref):
    out_ref[...] = (
        jnp.dot(h_ref[...], w_ref[...], preferred_element_type=jnp.float32)
        * norm_ref[...]
    )


def _matmul_norm(h, W, norm):
    nb = 10
    bs = N_NODES // nb
    return pl.pallas_call(
        _matmul_norm_kernel,
        grid=(nb,),
        in_specs=[
            pl.BlockSpec((bs, D), lambda i: (i, 0)),
            pl.BlockSpec((D, D), lambda i: (0, 0)),
            pl.BlockSpec((bs, 1), lambda i: (i, 0)),
        ],
        out_specs=pl.BlockSpec((bs, D), lambda i: (i, 0)),
        out_shape=jax.ShapeDtypeStruct((N_NODES, D), jnp.float32),
    )(h, W, norm)


def _finish_kernel(p0_ref, p1_ref, b_ref, out_ref):
    out_ref[...] = jnp.maximum(p0_ref[...] + p1_ref[...] + b_ref[...], 0.0)


def _finish(p0, p1, b):
    nb = 10
    bs = N_NODES // nb
    return pl.pallas_call(
        _finish_kernel,
        grid=(nb,),
        in_specs=[
            pl.BlockSpec((bs, D), lambda i: (i, 0)),
            pl.BlockSpec((bs, D), lambda i: (i, 0)),
            pl.BlockSpec((1, D), lambda i: (0, 0)),
        ],
        out_specs=pl.BlockSpec((bs, D), lambda i: (i, 0)),
        out_shape=jax.ShapeDtypeStruct((N_NODES, D), jnp.float32),
    )(p0, p1, b)


def _sc_scatter(normh, src_r, dst_r, zeros):
    mesh = plsc.VectorSubcoreMesh(core_axis_name="c", subcore_axis_name="s")

    @functools.partial(
        pl.kernel,
        mesh=mesh,
        out_type=jax.ShapeDtypeStruct((NC, N_PAD, D), jnp.float32),
        scratch_types=[
            pltpu.VMEM((NCHUNK * CHUNK,), jnp.int32),
            pltpu.VMEM((NCHUNK, CHUNK), jnp.int32),
            [pltpu.VMEM((CHUNK, D), jnp.float32) for _ in range(NBUF)],
            pltpu.VMEM_SHARED((N_PAD, D), jnp.float32),
            [pltpu.SemaphoreType.DMA for _ in range(NBUF)],
            [pltpu.SemaphoreType.DMA for _ in range(NBUF)],
        ],
    )
    def k(normh_hbm, src_hbm, dst_hbm, zeros_hbm, out_hbm,
          src_v, dst_v, rows, acc, gsem, ssem):
        cid = lax.axis_index("c")
        sid = lax.axis_index("s")
        wid = cid * NS + sid

        pltpu.sync_copy(src_hbm.at[wid], src_v)
        pltpu.sync_copy(dst_hbm.at[wid], dst_v)
        pltpu.sync_copy(
            zeros_hbm.at[pl.ds(sid * ROWS_PER_TILE, ROWS_PER_TILE)],
            acc.at[pl.ds(sid * ROWS_PER_TILE, ROWS_PER_TILE)],
        )
        plsc.subcore_barrier()

        def gather_start(j, b):
            pltpu.async_copy(normh_hbm.at[src_v.at[pl.ds(j * CHUNK, CHUNK)]],
                             rows[b], gsem[b])

        def gather_wait(j, b):
            pltpu.make_async_copy(
                normh_hbm.at[src_v.at[pl.ds(j * CHUNK, CHUNK)]], rows[b],
                gsem[b]).wait()

        def scatter_start(j, b):
            pltpu.async_copy(rows[b], acc.at[dst_v.at[j]], ssem[b], add=True)

        def scatter_wait(j, b):
            pltpu.make_async_copy(rows[b], acc.at[dst_v.at[j]],
                                  ssem[b]).wait()

        gather_start(0, 0)
        gather_wait(0, 0)
        scatter_start(0, 0)
        gather_start(1, 1)

        def steady(i, carry):
            for u in range(2):
                j = 2 * i - 1 + u
                b = 1 - u
                gather_wait(j, b)
                scatter_start(j, b)
                scatter_wait(j - 1, 1 - b)
                gather_start(j + 1, 1 - b)
            return carry

        lax.fori_loop(1, (NCHUNK - 2) // 2 + 1, steady, 0)

        jf = NCHUNK - 1
        gather_wait(jf, jf % 2)
        scatter_start(jf, jf % 2)
        scatter_wait(jf - 1, 1 - jf % 2)
        scatter_wait(jf, jf % 2)

        plsc.subcore_barrier()
        pltpu.sync_copy(
            acc.at[pl.ds(sid * ROWS_PER_TILE, ROWS_PER_TILE)],
            out_hbm.at[cid, pl.ds(sid * ROWS_PER_TILE, ROWS_PER_TILE)],
        )

    return k(normh, src_r, dst_r, zeros)


def kernel(h, edge_index, norm, W, b):
    normh = _matmul_norm(h, W, norm)
    ppw = EDGES_PER_W - N_EDGES // NW
    src_w = edge_index[0].reshape(NW, N_EDGES // NW)
    dst_w = edge_index[1].reshape(NW, N_EDGES // NW)
    src_pad = jnp.zeros((NW, ppw), jnp.int32)
    dst_pad = jnp.broadcast_to(
        (N_NODES + jnp.arange(NW, dtype=jnp.int32))[:, None], (NW, ppw))
    src_r = jnp.concatenate([src_w, src_pad], axis=1)
    dst_r = jnp.concatenate([dst_w, dst_pad], axis=1).reshape(NW, NCHUNK, CHUNK)
    zeros = jnp.zeros((N_PAD, D), jnp.float32)
    partials = _sc_scatter(normh, src_r, dst_r, zeros)
    return _finish(partials[0, :N_NODES], partials[1, :N_NODES], b.reshape(1, D))


# speedup vs baseline: 1.5198x; 1.0239x over previous
import functools

import jax
import jax.numpy as jnp
from jax import lax
from jax.experimental import pallas as pl
from jax.experimental.pallas import tpu as pltpu
from jax.experimental.pallas import tpu_sc as plsc

N_NODES = 10000
N_EDGES = 320000
D = 128

NC = 2
NS = 16
NW = NC * NS

CHUNK = 80
NCHUNK = 126
EDGES_PER_W = NCHUNK * CHUNK
NBUF = 2

N_PAD = 10240
ROWS_PER_TILE = N_PAD // NS


def _matmul_norm_kernel(h_ref, w_ref, norm_ref, out_ref):
    out_ref[...] = (
        jnp.dot(h_ref[...], w_ref[...], preferred_element_type=jnp.float32)
        * norm_ref[...]
    )


def _matmul_norm(h, W, norm):
    nb = 10
    bs = N_NODES // nb
    return pl.pallas_call(
        _matmul_norm_kernel,
        grid=(nb,),
        in_specs=[
            pl.BlockSpec((bs, D), lambda i: (i, 0)),
            pl.BlockSpec((D, D), lambda i: (0, 0)),
            pl.BlockSpec((bs, 1), lambda i: (i, 0)),
        ],
        out_specs=pl.BlockSpec((bs, D), lambda i: (i, 0)),
        out_shape=jax.ShapeDtypeStruct((N_NODES, D), jnp.float32),
    )(h, W, norm)


def _finish_kernel(p_ref, b_ref, out_ref):
    out_ref[...] = jnp.maximum(p_ref[0] + p_ref[1] + b_ref[...], 0.0)


def _finish(partials, b):
    nb = 10
    bs = N_NODES // nb
    return pl.pallas_call(
        _finish_kernel,
        grid=(nb,),
        in_specs=[
            pl.BlockSpec((NC, bs, D), lambda i: (0, i, 0)),
            pl.BlockSpec((1, D), lambda i: (0, 0)),
        ],
        out_specs=pl.BlockSpec((bs, D), lambda i: (i, 0)),
        out_shape=jax.ShapeDtypeStruct((N_NODES, D), jnp.float32),
    )(partials, b)


def _sc_scatter(normh, src_r, dst_r, zeros):
    mesh = plsc.VectorSubcoreMesh(core_axis_name="c", subcore_axis_name="s")

    @functools.partial(
        pl.kernel,
        mesh=mesh,
        out_type=jax.ShapeDtypeStruct((NC, N_PAD, D), jnp.float32),
        scratch_types=[
            pltpu.VMEM((NCHUNK * CHUNK,), jnp.int32),
            pltpu.VMEM((NCHUNK, CHUNK), jnp.int32),
            [pltpu.VMEM((CHUNK, D), jnp.float32) for _ in range(NBUF)],
            pltpu.VMEM_SHARED((N_PAD, D), jnp.float32),
            [pltpu.SemaphoreType.DMA for _ in range(NBUF)],
            [pltpu.SemaphoreType.DMA for _ in range(NBUF)],
        ],
    )
    def k(normh_hbm, src_hbm, dst_hbm, zeros_hbm, out_hbm,
          src_v, dst_v, rows, acc, gsem, ssem):
        cid = lax.axis_index("c")
        sid = lax.axis_index("s")
        wid = cid * NS + sid

        pltpu.sync_copy(src_hbm.at[wid], src_v)
        pltpu.sync_copy(dst_hbm.at[wid], dst_v)
        pltpu.sync_copy(
            zeros_hbm.at[pl.ds(sid * ROWS_PER_TILE, ROWS_PER_TILE)],
            acc.at[pl.ds(sid * ROWS_PER_TILE, ROWS_PER_TILE)],
        )
        plsc.subcore_barrier()

        def gather_start(j, b):
            pltpu.async_copy(normh_hbm.at[src_v.at[pl.ds(j * CHUNK, CHUNK)]],
                             rows[b], gsem[b])

        def gather_wait(j, b):
            pltpu.make_async_copy(
                normh_hbm.at[src_v.at[pl.ds(j * CHUNK, CHUNK)]], rows[b],
                gsem[b]).wait()

        def scatter_start(j, b):
            pltpu.async_copy(rows[b], acc.at[dst_v.at[j]], ssem[b], add=True)

        def scatter_wait(j, b):
            pltpu.make_async_copy(rows[b], acc.at[dst_v.at[j]],
                                  ssem[b]).wait()

        gather_start(0, 0)
        gather_wait(0, 0)
        scatter_start(0, 0)
        gather_start(1, 1)

        def steady(i, carry):
            for u in range(2):
                j = 2 * i - 1 + u
                b = 1 - u
                gather_wait(j, b)
                scatter_start(j, b)
                scatter_wait(j - 1, 1 - b)
                gather_start(j + 1, 1 - b)
            return carry

        lax.fori_loop(1, (NCHUNK - 2) // 2 + 1, steady, 0)

        jf = NCHUNK - 1
        gather_wait(jf, jf % 2)
        scatter_start(jf, jf % 2)
        scatter_wait(jf - 1, 1 - jf % 2)
        scatter_wait(jf, jf % 2)

        plsc.subcore_barrier()
        pltpu.sync_copy(
            acc.at[pl.ds(sid * ROWS_PER_TILE, ROWS_PER_TILE)],
            out_hbm.at[cid, pl.ds(sid * ROWS_PER_TILE, ROWS_PER_TILE)],
        )

    return k(normh, src_r, dst_r, zeros)


def kernel(h, edge_index, norm, W, b):
    normh = _matmul_norm(h, W, norm)
    ppw = EDGES_PER_W - N_EDGES // NW
    src_w = edge_index[0].reshape(NW, N_EDGES // NW)
    dst_w = edge_index[1].reshape(NW, N_EDGES // NW)
    src_pad = jnp.zeros((NW, ppw), jnp.int32)
    dst_pad = jnp.broadcast_to(
        (N_NODES + jnp.arange(NW, dtype=jnp.int32))[:, None], (NW, ppw))
    src_r = jnp.concatenate([src_w, src_pad], axis=1)
    dst_r = jnp.concatenate([dst_w, dst_pad], axis=1).reshape(NW, NCHUNK, CHUNK)
    zeros = jnp.zeros((N_PAD, D), jnp.float32)
    partials = _sc_scatter(normh, src_r, dst_r, zeros)
    return _finish(partials, b.reshape(1, D))


# speedup vs baseline: 2.1636x; 1.4237x over previous
import functools

import jax
import jax.numpy as jnp
from jax import lax
from jax.experimental import pallas as pl
from jax.experimental.pallas import tpu as pltpu
from jax.experimental.pallas import tpu_sc as plsc

N_NODES = 10000
N_EDGES = 320000
D = 128

NC = 2
NS = 16
NW = NC * NS

CHUNK = 80
NCHUNK = 125
EDGES_PER_W = NCHUNK * CHUNK
NBUF = 2

N_PAD = 10240
ROWS_PER_TILE = N_PAD // NS


def _matmul_norm_kernel(h_ref, w_ref, norm_ref, out_ref, z_ref):
    out_ref[...] = (
        jnp.dot(h_ref[...], w_ref[...], preferred_element_type=jnp.float32)
        * norm_ref[...]
    )
    z_ref[...] = jnp.zeros_like(z_ref)


def _matmul_norm(h, W, norm):
    nb = 10
    bs = N_NODES // nb
    zbs = N_PAD // nb
    return pl.pallas_call(
        _matmul_norm_kernel,
        grid=(nb,),
        in_specs=[
            pl.BlockSpec((bs, D), lambda i: (i, 0)),
            pl.BlockSpec((D, D), lambda i: (0, 0)),
            pl.BlockSpec((bs, 1), lambda i: (i, 0)),
        ],
        out_specs=[
            pl.BlockSpec((bs, D), lambda i: (i, 0)),
            pl.BlockSpec((zbs, D), lambda i: (i, 0)),
        ],
        out_shape=[
            jax.ShapeDtypeStruct((N_NODES, D), jnp.float32),
            jax.ShapeDtypeStruct((N_PAD, D), jnp.float32),
        ],
    )(h, W, norm)


def _finish_kernel(p_ref, b_ref, out_ref):
    out_ref[...] = jnp.maximum(p_ref[0] + p_ref[1] + b_ref[...], 0.0)


def _finish(partials, b):
    nb = 10
    bs = N_NODES // nb
    return pl.pallas_call(
        _finish_kernel,
        grid=(nb,),
        in_specs=[
            pl.BlockSpec((NC, bs, D), lambda i: (0, i, 0)),
            pl.BlockSpec((1, D), lambda i: (0, 0)),
        ],
        out_specs=pl.BlockSpec((bs, D), lambda i: (i, 0)),
        out_shape=jax.ShapeDtypeStruct((N_NODES, D), jnp.float32),
    )(partials, b)


def _sc_scatter(normh, src_r, dst_r, zeros):
    mesh = plsc.VectorSubcoreMesh(core_axis_name="c", subcore_axis_name="s")

    @functools.partial(
        pl.kernel,
        mesh=mesh,
        out_type=jax.ShapeDtypeStruct((NC, N_PAD, D), jnp.float32),
        scratch_types=[
            pltpu.VMEM((NCHUNK * CHUNK,), jnp.int32),
            pltpu.VMEM((NCHUNK, CHUNK), jnp.int32),
            [pltpu.VMEM((CHUNK, D), jnp.float32) for _ in range(NBUF)],
            pltpu.VMEM_SHARED((N_PAD, D), jnp.float32),
            [pltpu.SemaphoreType.DMA for _ in range(NBUF)],
            [pltpu.SemaphoreType.DMA for _ in range(NBUF)],
        ],
    )
    def k(normh_hbm, src_hbm, dst_hbm, zeros_hbm, out_hbm,
          src_v, dst_v, rows, acc, gsem, ssem):
        cid = lax.axis_index("c")
        sid = lax.axis_index("s")
        wid = cid * NS + sid

        pltpu.sync_copy(src_hbm.at[wid], src_v)
        pltpu.sync_copy(dst_hbm.at[wid], dst_v)
        pltpu.sync_copy(
            zeros_hbm.at[pl.ds(sid * ROWS_PER_TILE, ROWS_PER_TILE)],
            acc.at[pl.ds(sid * ROWS_PER_TILE, ROWS_PER_TILE)],
        )
        plsc.subcore_barrier()

        def gather_start(j, b):
            pltpu.async_copy(normh_hbm.at[src_v.at[pl.ds(j * CHUNK, CHUNK)]],
                             rows[b], gsem[b])

        def gather_wait(j, b):
            pltpu.make_async_copy(
                normh_hbm.at[src_v.at[pl.ds(j * CHUNK, CHUNK)]], rows[b],
                gsem[b]).wait()

        def scatter_start(j, b):
            pltpu.async_copy(rows[b], acc.at[dst_v.at[j]], ssem[b], add=True)

        def scatter_wait(j, b):
            pltpu.make_async_copy(rows[b], acc.at[dst_v.at[j]],
                                  ssem[b]).wait()

        gather_start(0, 0)
        gather_wait(0, 0)
        scatter_start(0, 0)
        gather_start(1, 1)

        def steady(i, carry):
            for u in range(2):
                j = 2 * i - 1 + u
                b = 1 - u
                gather_wait(j, b)
                scatter_start(j, b)
                scatter_wait(j - 1, 1 - b)
                gather_start(j + 1, 1 - b)
            return carry

        nsteady = (NCHUNK - 2) // 2
        lax.fori_loop(1, nsteady + 1, steady, 0)

        for j in range(2 * nsteady + 1, NCHUNK):
            b = j % 2
            gather_wait(j, b)
            scatter_start(j, b)
            scatter_wait(j - 1, 1 - b)
            if j + 1 < NCHUNK:
                gather_start(j + 1, 1 - b)
        scatter_wait(NCHUNK - 1, (NCHUNK - 1) % 2)

        plsc.subcore_barrier()
        pltpu.sync_copy(
            acc.at[pl.ds(sid * ROWS_PER_TILE, ROWS_PER_TILE)],
            out_hbm.at[cid, pl.ds(sid * ROWS_PER_TILE, ROWS_PER_TILE)],
        )

    return k(normh, src_r, dst_r, zeros)


def kernel(h, edge_index, norm, W, b):
    normh, zeros = _matmul_norm(h, W, norm)
    src_r = edge_index[0].reshape(NW, EDGES_PER_W)
    dst_r = edge_index[1].reshape(NW, NCHUNK, CHUNK)
    partials = _sc_scatter(normh, src_r, dst_r, zeros)
    return _finish(partials, b.reshape(1, D))


# speedup vs baseline: 2.3670x; 1.0940x over previous
import functools

import jax
import jax.numpy as jnp
from jax import lax
from jax.experimental import pallas as pl
from jax.experimental.pallas import tpu as pltpu
from jax.experimental.pallas import tpu_sc as plsc

N_NODES = 10000
N_EDGES = 320000
D = 128

NC = 2
NS = 16
NW = NC * NS

CHUNK = 40
NCHUNK = 250
EDGES_PER_W = NCHUNK * CHUNK
NBUF = 5
GD = 2

N_PAD = 10240
ROWS_PER_TILE = N_PAD // NS


def _matmul_norm_kernel(h_ref, w_ref, norm_ref, out_ref, z_ref):
    out_ref[...] = (
        jnp.dot(h_ref[...], w_ref[...], preferred_element_type=jnp.float32)
        * norm_ref[...]
    )
    z_ref[...] = jnp.zeros_like(z_ref)


def _matmul_norm(h, W, norm):
    nb = 10
    bs = N_NODES // nb
    zbs = N_PAD // nb
    return pl.pallas_call(
        _matmul_norm_kernel,
        grid=(nb,),
        in_specs=[
            pl.BlockSpec((bs, D), lambda i: (i, 0)),
            pl.BlockSpec((D, D), lambda i: (0, 0)),
            pl.BlockSpec((bs, 1), lambda i: (i, 0)),
        ],
        out_specs=[
            pl.BlockSpec((bs, D), lambda i: (i, 0)),
            pl.BlockSpec((zbs, D), lambda i: (i, 0)),
        ],
        out_shape=[
            jax.ShapeDtypeStruct((N_NODES, D), jnp.float32),
            jax.ShapeDtypeStruct((N_PAD, D), jnp.float32),
        ],
    )(h, W, norm)


def _finish_kernel(p_ref, b_ref, out_ref):
    out_ref[...] = jnp.maximum(p_ref[0] + p_ref[1] + b_ref[...], 0.0)


def _finish(partials, b):
    nb = 10
    bs = N_NODES // nb
    return pl.pallas_call(
        _finish_kernel,
        grid=(nb,),
        in_specs=[
            pl.BlockSpec((NC, bs, D), lambda i: (0, i, 0)),
            pl.BlockSpec((1, D), lambda i: (0, 0)),
        ],
        out_specs=pl.BlockSpec((bs, D), lambda i: (i, 0)),
        out_shape=jax.ShapeDtypeStruct((N_NODES, D), jnp.float32),
    )(partials, b)


def _sc_scatter(normh, src_r, dst_r, zeros):
    mesh = plsc.VectorSubcoreMesh(core_axis_name="c", subcore_axis_name="s")

    @functools.partial(
        pl.kernel,
        mesh=mesh,
        out_type=jax.ShapeDtypeStruct((NC, N_PAD, D), jnp.float32),
        scratch_types=[
            pltpu.VMEM((NCHUNK * CHUNK,), jnp.int32),
            pltpu.VMEM((NCHUNK * CHUNK,), jnp.int32),
            [pltpu.VMEM((CHUNK, D), jnp.float32) for _ in range(NBUF)],
            pltpu.VMEM_SHARED((N_PAD, D), jnp.float32),
            [pltpu.SemaphoreType.DMA for _ in range(NBUF)],
            [pltpu.SemaphoreType.DMA for _ in range(NBUF)],
        ],
    )
    def k(normh_hbm, src_hbm, dst_hbm, zeros_hbm, out_hbm,
          src_v, dst_v, rows, acc, gsem, ssem):
        cid = lax.axis_index("c")
        sid = lax.axis_index("s")
        wid = cid * NS + sid

        pltpu.sync_copy(src_hbm.at[wid], src_v)
        pltpu.sync_copy(dst_hbm.at[wid], dst_v)
        pltpu.sync_copy(
            zeros_hbm.at[pl.ds(sid * ROWS_PER_TILE, ROWS_PER_TILE)],
            acc.at[pl.ds(sid * ROWS_PER_TILE, ROWS_PER_TILE)],
        )
        plsc.subcore_barrier()

        def gather_start(j, b):
            pltpu.async_copy(normh_hbm.at[src_v.at[pl.ds(j * CHUNK, CHUNK)]],
                             rows[b], gsem[b])

        def gather_wait(j, b):
            pltpu.make_async_copy(
                normh_hbm.at[src_v.at[pl.ds(j * CHUNK, CHUNK)]], rows[b],
                gsem[b]).wait()

        def scatter_start(j, b):
            pltpu.async_copy(rows[b], acc.at[dst_v.at[pl.ds(j * CHUNK, CHUNK)]],
                             ssem[b], add=True)

        def scatter_wait(j, b):
            pltpu.make_async_copy(rows[b],
                                  acc.at[dst_v.at[pl.ds(j * CHUNK, CHUNK)]],
                                  ssem[b]).wait()

        gather_start(0, 0)
        gather_start(1, 1)
        for j in range(NBUF):
            gather_wait(j, j)
            scatter_start(j, j)
            if j >= NBUF - GD:
                scatter_wait(j - (NBUF - GD), (j - (NBUF - GD)) % NBUF)
            gather_start(j + GD, (j + GD) % NBUF)

        def steady(i, carry):
            for u in range(NBUF):
                j = i * NBUF + u
                gather_wait(j, u)
                scatter_start(j, u)
                scatter_wait(j - (NBUF - GD), (u - (NBUF - GD)) % NBUF)
                gather_start(j + GD, (u + GD) % NBUF)
            return carry

        lax.fori_loop(1, NCHUNK // NBUF - 1, steady, 0)

        for j in range(NCHUNK - NBUF, NCHUNK):
            u = j % NBUF
            gather_wait(j, u)
            scatter_start(j, u)
            scatter_wait(j - (NBUF - GD), (u - (NBUF - GD)) % NBUF)
            if j + GD < NCHUNK:
                gather_start(j + GD, (u + GD) % NBUF)
        for j in range(NCHUNK - (NBUF - GD), NCHUNK):
            scatter_wait(j, j % NBUF)

        plsc.subcore_barrier()
        pltpu.sync_copy(
            acc.at[pl.ds(sid * ROWS_PER_TILE, ROWS_PER_TILE)],
            out_hbm.at[cid, pl.ds(sid * ROWS_PER_TILE, ROWS_PER_TILE)],
        )

    return k(normh, src_r, dst_r, zeros)


def kernel(h, edge_index, norm, W, b):
    normh, zeros = _matmul_norm(h, W, norm)
    src_r = edge_index[0].reshape(NW, EDGES_PER_W)
    dst_r = edge_index[1].reshape(NW, EDGES_PER_W)
    partials = _sc_scatter(normh, src_r, dst_r, zeros)
    return _finish(partials, b.reshape(1, D))


# speedup vs baseline: 2.8074x; 1.1861x over previous
import functools

import jax
import jax.numpy as jnp
from jax import lax
from jax.experimental import pallas as pl
from jax.experimental.pallas import tpu as pltpu
from jax.experimental.pallas import tpu_sc as plsc

N_NODES = 10000
N_EDGES = 320000
D = 128

NC = 2
NS = 16
NW = NC * NS

CHUNK = 40
NCHUNK = 250
EDGES_PER_W = NCHUNK * CHUNK
NBUF = 5
GD = 3

N_PAD = 10240
ROWS_PER_TILE = N_PAD // NS


def _matmul_norm_kernel(h_ref, w_ref, norm_ref, out_ref, z_ref):
    out_ref[...] = (
        jnp.dot(h_ref[...], w_ref[...], preferred_element_type=jnp.float32)
        * norm_ref[...]
    )
    z_ref[...] = jnp.zeros_like(z_ref)


def _matmul_norm(h, W, norm):
    nb = 10
    bs = N_NODES // nb
    zbs = N_PAD // nb
    return pl.pallas_call(
        _matmul_norm_kernel,
        grid=(nb,),
        in_specs=[
            pl.BlockSpec((bs, D), lambda i: (i, 0)),
            pl.BlockSpec((D, D), lambda i: (0, 0)),
            pl.BlockSpec((bs, 1), lambda i: (i, 0)),
        ],
        out_specs=[
            pl.BlockSpec((bs, D), lambda i: (i, 0)),
            pl.BlockSpec((zbs, D), lambda i: (i, 0)),
        ],
        out_shape=[
            jax.ShapeDtypeStruct((N_NODES, D), jnp.float32),
            jax.ShapeDtypeStruct((N_PAD, D), jnp.float32),
        ],
    )(h, W, norm)


def _finish_kernel(p_ref, b_ref, out_ref):
    out_ref[...] = jnp.maximum(p_ref[0] + p_ref[1] + b_ref[...], 0.0)


def _finish(partials, b):
    nb = 10
    bs = N_NODES // nb
    return pl.pallas_call(
        _finish_kernel,
        grid=(nb,),
        in_specs=[
            pl.BlockSpec((NC, bs, D), lambda i: (0, i, 0)),
            pl.BlockSpec((1, D), lambda i: (0, 0)),
        ],
        out_specs=pl.BlockSpec((bs, D), lambda i: (i, 0)),
        out_shape=jax.ShapeDtypeStruct((N_NODES, D), jnp.float32),
    )(partials, b)


def _sc_scatter(normh, src_r, dst_r, zeros):
    mesh = plsc.VectorSubcoreMesh(core_axis_name="c", subcore_axis_name="s")

    @functools.partial(
        pl.kernel,
        mesh=mesh,
        out_type=jax.ShapeDtypeStruct((NC, N_PAD, D), jnp.float32),
        scratch_types=[
            pltpu.VMEM((NCHUNK * CHUNK,), jnp.int32),
            pltpu.VMEM((NCHUNK * CHUNK,), jnp.int32),
            [pltpu.VMEM((CHUNK, D), jnp.float32) for _ in range(NBUF)],
            pltpu.VMEM_SHARED((N_PAD, D), jnp.float32),
            [pltpu.SemaphoreType.DMA for _ in range(NBUF)],
            [pltpu.SemaphoreType.DMA for _ in range(NBUF)],
        ],
    )
    def k(normh_hbm, src_hbm, dst_hbm, zeros_hbm, out_hbm,
          src_v, dst_v, rows, acc, gsem, ssem):
        cid = lax.axis_index("c")
        sid = lax.axis_index("s")
        wid = cid * NS + sid

        pltpu.sync_copy(src_hbm.at[wid], src_v)
        pltpu.sync_copy(dst_hbm.at[wid], dst_v)
        pltpu.sync_copy(
            zeros_hbm.at[pl.ds(sid * ROWS_PER_TILE, ROWS_PER_TILE)],
            acc.at[pl.ds(sid * ROWS_PER_TILE, ROWS_PER_TILE)],
        )
        plsc.subcore_barrier()

        def gather_start(j, b):
            pltpu.async_copy(normh_hbm.at[src_v.at[pl.ds(j * CHUNK, CHUNK)]],
                             rows[b], gsem[b])

        def gather_wait(j, b):
            pltpu.make_async_copy(
                normh_hbm.at[src_v.at[pl.ds(j * CHUNK, CHUNK)]], rows[b],
                gsem[b]).wait()

        def scatter_start(j, b):
            pltpu.async_copy(rows[b], acc.at[dst_v.at[pl.ds(j * CHUNK, CHUNK)]],
                             ssem[b], add=True)

        def scatter_wait(j, b):
            pltpu.make_async_copy(rows[b],
                                  acc.at[dst_v.at[pl.ds(j * CHUNK, CHUNK)]],
                                  ssem[b]).wait()

        for j in range(GD):
            gather_start(j, j)
        for j in range(NBUF):
            gather_wait(j, j)
            scatter_start(j, j)
            if j >= NBUF - GD:
                scatter_wait(j - (NBUF - GD), (j - (NBUF - GD)) % NBUF)
            gather_start(j + GD, (j + GD) % NBUF)

        def steady(i, carry):
            for u in range(NBUF):
                j = i * NBUF + u
                gather_wait(j, u)
                scatter_start(j, u)
                scatter_wait(j - (NBUF - GD), (u - (NBUF - GD)) % NBUF)
                gather_start(j + GD, (u + GD) % NBUF)
            return carry

        lax.fori_loop(1, NCHUNK // NBUF - 1, steady, 0)

        for j in range(NCHUNK - NBUF, NCHUNK):
            u = j % NBUF
            gather_wait(j, u)
            scatter_start(j, u)
            scatter_wait(j - (NBUF - GD), (u - (NBUF - GD)) % NBUF)
            if j + GD < NCHUNK:
                gather_start(j + GD, (u + GD) % NBUF)
        for j in range(NCHUNK - (NBUF - GD), NCHUNK):
            scatter_wait(j, j % NBUF)

        plsc.subcore_barrier()
        pltpu.sync_copy(
            acc.at[pl.ds(sid * ROWS_PER_TILE, ROWS_PER_TILE)],
            out_hbm.at[cid, pl.ds(sid * ROWS_PER_TILE, ROWS_PER_TILE)],
        )

    return k(normh, src_r, dst_r, zeros)


def kernel(h, edge_index, norm, W, b):
    normh, zeros = _matmul_norm(h, W, norm)
    src_r = edge_index[0].reshape(NW, EDGES_PER_W)
    dst_r = edge_index[1].reshape(NW, EDGES_PER_W)
    partials = _sc_scatter(normh, src_r, dst_r, zeros)
    return _finish(partials, b.reshape(1, D))


# speedup vs baseline: 2.9896x; 1.0649x over previous
import functools

import jax
import jax.numpy as jnp
from jax import lax
from jax.experimental import pallas as pl
from jax.experimental.pallas import tpu as pltpu
from jax.experimental.pallas import tpu_sc as plsc

N_NODES = 10000
N_EDGES = 320000
D = 128

NC = 2
NS = 16
NW = NC * NS

CHUNK = 40
NCHUNK = 250
EDGES_PER_W = NCHUNK * CHUNK
NBUF = 5
GD = 4

N_PAD = 10240
ROWS_PER_TILE = N_PAD // NS


def _matmul_norm_kernel(h_ref, w_ref, norm_ref, out_ref, z_ref):
    out_ref[...] = (
        jnp.dot(h_ref[...], w_ref[...], preferred_element_type=jnp.float32)
        * norm_ref[...]
    )
    z_ref[...] = jnp.zeros_like(z_ref)


def _matmul_norm(h, W, norm):
    nb = 10
    bs = N_NODES // nb
    zbs = N_PAD // nb
    return pl.pallas_call(
        _matmul_norm_kernel,
        grid=(nb,),
        in_specs=[
            pl.BlockSpec((bs, D), lambda i: (i, 0)),
            pl.BlockSpec((D, D), lambda i: (0, 0)),
            pl.BlockSpec((bs, 1), lambda i: (i, 0)),
        ],
        out_specs=[
            pl.BlockSpec((bs, D), lambda i: (i, 0)),
            pl.BlockSpec((zbs, D), lambda i: (i, 0)),
        ],
        out_shape=[
            jax.ShapeDtypeStruct((N_NODES, D), jnp.float32),
            jax.ShapeDtypeStruct((N_PAD, D), jnp.float32),
        ],
    )(h, W, norm)


def _finish_kernel(p_ref, b_ref, out_ref):
    out_ref[...] = jnp.maximum(p_ref[0] + p_ref[1] + b_ref[...], 0.0)


def _finish(partials, b):
    nb = 10
    bs = N_NODES // nb
    return pl.pallas_call(
        _finish_kernel,
        grid=(nb,),
        in_specs=[
            pl.BlockSpec((NC, bs, D), lambda i: (0, i, 0)),
            pl.BlockSpec((1, D), lambda i: (0, 0)),
        ],
        out_specs=pl.BlockSpec((bs, D), lambda i: (i, 0)),
        out_shape=jax.ShapeDtypeStruct((N_NODES, D), jnp.float32),
    )(partials, b)


def _sc_scatter(normh, src_r, dst_r, zeros):
    mesh = plsc.VectorSubcoreMesh(core_axis_name="c", subcore_axis_name="s")

    @functools.partial(
        pl.kernel,
        mesh=mesh,
        out_type=jax.ShapeDtypeStruct((NC, N_PAD, D), jnp.float32),
        scratch_types=[
            pltpu.VMEM((NCHUNK * CHUNK,), jnp.int32),
            pltpu.VMEM((NCHUNK * CHUNK,), jnp.int32),
            [pltpu.VMEM((CHUNK, D), jnp.float32) for _ in range(NBUF)],
            pltpu.VMEM_SHARED((N_PAD, D), jnp.float32),
            [pltpu.SemaphoreType.DMA for _ in range(NBUF)],
            [pltpu.SemaphoreType.DMA for _ in range(NBUF)],
        ],
    )
    def k(normh_hbm, src_hbm, dst_hbm, zeros_hbm, out_hbm,
          src_v, dst_v, rows, acc, gsem, ssem):
        cid = lax.axis_index("c")
        sid = lax.axis_index("s")
        wid = cid * NS + sid

        pltpu.sync_copy(src_hbm.at[wid], src_v)
        pltpu.sync_copy(dst_hbm.at[wid], dst_v)
        pltpu.sync_copy(
            zeros_hbm.at[pl.ds(sid * ROWS_PER_TILE, ROWS_PER_TILE)],
            acc.at[pl.ds(sid * ROWS_PER_TILE, ROWS_PER_TILE)],
        )
        plsc.subcore_barrier()

        def gather_start(j, b):
            pltpu.async_copy(normh_hbm.at[src_v.at[pl.ds(j * CHUNK, CHUNK)]],
                             rows[b], gsem[b])

        def gather_wait(j, b):
            pltpu.make_async_copy(
                normh_hbm.at[src_v.at[pl.ds(j * CHUNK, CHUNK)]], rows[b],
                gsem[b]).wait()

        def scatter_start(j, b):
            pltpu.async_copy(rows[b], acc.at[dst_v.at[pl.ds(j * CHUNK, CHUNK)]],
                             ssem[b], add=True)

        def scatter_wait(j, b):
            pltpu.make_async_copy(rows[b],
                                  acc.at[dst_v.at[pl.ds(j * CHUNK, CHUNK)]],
                                  ssem[b]).wait()

        for j in range(GD):
            gather_start(j, j)
        for j in range(NBUF):
            gather_wait(j, j)
            scatter_start(j, j)
            if j >= NBUF - GD:
                scatter_wait(j - (NBUF - GD), (j - (NBUF - GD)) % NBUF)
            gather_start(j + GD, (j + GD) % NBUF)

        def steady(i, carry):
            for u in range(NBUF):
                j = i * NBUF + u
                gather_wait(j, u)
                scatter_start(j, u)
                scatter_wait(j - (NBUF - GD), (u - (NBUF - GD)) % NBUF)
                gather_start(j + GD, (u + GD) % NBUF)
            return carry

        lax.fori_loop(1, NCHUNK // NBUF - 1, steady, 0)

        for j in range(NCHUNK - NBUF, NCHUNK):
            u = j % NBUF
            gather_wait(j, u)
            scatter_start(j, u)
            scatter_wait(j - (NBUF - GD), (u - (NBUF - GD)) % NBUF)
            if j + GD < NCHUNK:
                gather_start(j + GD, (u + GD) % NBUF)
        for j in range(NCHUNK - (NBUF - GD), NCHUNK):
            scatter_wait(j, j % NBUF)

        plsc.subcore_barrier()
        pltpu.sync_copy(
            acc.at[pl.ds(sid * ROWS_PER_TILE, ROWS_PER_TILE)],
            out_hbm.at[cid, pl.ds(sid * ROWS_PER_TILE, ROWS_PER_TILE)],
        )

    return k(normh, src_r, dst_r, zeros)


def kernel(h, edge_index, norm, W, b):
    normh, zeros = _matmul_norm(h, W, norm)
    src_r = edge_index[0].reshape(NW, EDGES_PER_W)
    dst_r = edge_index[1].reshape(NW, EDGES_PER_W)
    partials = _sc_scatter(normh, src_r, dst_r, zeros)
    return _finish(partials, b.reshape(1, D))
